# baseline (speedup 1.0000x reference)
import functools

import jax
import jax.numpy as jnp
from jax import lax
from jax.experimental import pallas as pl
from jax.experimental.pallas import tpu as pltpu
from jax.experimental.pallas import tpu_sc as plsc

BATCH = 16384
NENT = 1000000
DIM = 64
HALF = DIM // 2
LANES = 16
NREL = 230
NW = 32
RANGE = 31232
ALIGNED_END = 999936
NTAIL = NENT - ALIGNED_END
E = 384
NCHUNK = 84
MAXSTART = ALIGNED_END - E
LISTCAP = 1792
PACK = 48
NSTAGE = 2 * BATCH
STAGE = NSTAGE + PACK

BPW = BATCH // NW
NCH2 = 2
CH2 = BPW // NCH2
GROUPS2 = CH2 // LANES

_mesh = plsc.VectorSubcoreMesh(core_axis_name="c", subcore_axis_name="s")


@functools.partial(
    pl.kernel,
    mesh=_mesh,
    out_type=jax.ShapeDtypeStruct((STAGE, 2 * DIM), jnp.float32),
    scratch_types=[
        pltpu.VMEM((2048,), jnp.int32),
        pltpu.VMEM((LISTCAP,), jnp.int32),
        pltpu.VMEM((LISTCAP,), jnp.int32),
        [pltpu.VMEM((DIM, E), jnp.float32)] * 3,
        [pltpu.VMEM((PACK, 2 * DIM), jnp.float32)] * 3,
        [pltpu.VMEM((PACK,), jnp.int32)] * 3,
        pltpu.VMEM((PACK,), jnp.int32),
        pltpu.VMEM((NTAIL, DIM), jnp.float32),
        [pltpu.SemaphoreType.DMA] * 3,
        [pltpu.SemaphoreType.DMA] * 3,
    ],
    compiler_params=pltpu.CompilerParams(needs_layout_passes=False),
)
def _phase1(entT_hbm, tail_hbm, hidx_hbm, tidx_hbm, stage_hbm,
            piece, entlist, poslist, bufs, packs, poss, hitent, tailtab,
            semS, semW):
    wid = lax.axis_index("s") * 2 + lax.axis_index("c")
    lo = wid * RANGE
    hi = jnp.where(wid == NW - 1, NENT, lo + RANGE)
    lane_iota = lax.iota(jnp.int32, LANES)

    def chunk_start(c):
        return jnp.minimum(lo + c * E, MAXSTART)

    for k in range(3):
        pltpu.async_copy(
            entT_hbm.at[:, pl.ds(chunk_start(k), E)], bufs[k], semS[k])
    pltpu.sync_copy(tail_hbm, tailtab)

    def init_list(i, carry):
        entlist[pl.ds(i * LANES, LANES)] = jnp.full((LANES,), -1, jnp.int32)
        return carry
    lax.fori_loop(0, LISTCAP // LANES, init_list, 0)

    def scan_src(arr_hbm, pos0, cnt_in):
        def piece_body(p, cnt):
            pltpu.sync_copy(arr_hbm.at[pl.ds(p * 2048, 2048)], piece)

            def vec_body(i, cnt):
                v = piece[pl.ds(i * LANES, LANES)]
                m = (v >= lo) & (v < hi)
                cc = jnp.minimum(cnt, LISTCAP - LANES)
                plsc.store_compressed(entlist.at[pl.ds(cc, LANES)], v, mask=m)
                pos = pos0 + p * 2048 + i * LANES + lane_iota
                plsc.store_compressed(poslist.at[pl.ds(cc, LANES)], pos, mask=m)
                npop = plsc.all_reduce_population_count(m)
                return cnt + npop[0]

            return lax.fori_loop(0, 2048 // LANES, vec_body, cnt)
        return lax.fori_loop(0, BATCH // 2048, piece_body, cnt_in)

    cnt = scan_src(hidx_hbm, 0, jnp.int32(0))
    cnt = scan_src(tidx_hbm, BATCH, cnt)
    nvec = jnp.minimum((cnt + LANES - 1) // LANES, LISTCAP // LANES)

    def drain_stream(sem, buf):
        pltpu.make_async_copy(
            entT_hbm.at[:, pl.ds(0, E)], buf, sem).wait()

    def drain_scatter(sem, pack):
        pltpu.make_async_copy(
            stage_hbm.at[pl.ds(0, PACK), :], pack, sem).wait()

    def extract_chunk(e0, e1, col_of, src_gather, pack, posb, semWX):
        for q in range(PACK // LANES):
            posb[pl.ds(q * LANES, LANES)] = (
                NSTAGE + q * LANES + lane_iota)

        def list_body(i, hcnt):
            ev = entlist[pl.ds(i * LANES, LANES)]
            m = (ev >= e0) & (ev < e1)
            pv = poslist[pl.ds(i * LANES, LANES)]
            hc = jnp.minimum(hcnt, PACK - LANES)
            plsc.store_compressed(hitent.at[pl.ds(hc, LANES)], ev, mask=m)
            plsc.store_compressed(posb.at[pl.ds(hc, LANES)], pv, mask=m)
            return hcnt + plsc.all_reduce_population_count(m)[0]

        hcnt = lax.fori_loop(0, nvec, list_body, jnp.int32(0))
        hcnt = jnp.minimum(hcnt, PACK)

        def member_body(j, carry):
            jsplat = jnp.broadcast_to(j, (LANES,))
            e = plsc.load_gather(hitent, [jsplat])[0]
            col = col_of(e)
            for q in range(DIM // LANES):
                seg = src_gather(q, col)
                plsc.store_scatter(
                    pack, [jsplat, q * LANES + lane_iota], seg)
            return carry

        lax.fori_loop(0, hcnt, member_body, 0)
        pltpu.async_copy(pack, stage_hbm.at[posb], semWX)

    def triple_body(j, carry):
        for k in range(3):
            c = 3 * j + k
            e0 = lo + c * E
            s0 = chunk_start(c)
            drain_stream(semS[k], bufs[k])

            @pl.when(j > 0)
            def _():
                drain_scatter(semW[k], packs[k])

            def gk(q, col, _buf=bufs[k]):
                return plsc.load_gather(
                    _buf,
                    [q * LANES + lane_iota, jnp.broadcast_to(col, (LANES,))])

            extract_chunk(e0, e0 + E, lambda e: e - s0, gk,
                          packs[k], poss[k], semW[k])
            pltpu.async_copy(
                entT_hbm.at[:, pl.ds(chunk_start(c + 3), E)],
                bufs[k], semS[k])
        return carry

    lax.fori_loop(0, NCHUNK // 3, triple_body, 0)

    drain_scatter(semW[0], packs[0])

    def gatherT(q, row):
        return plsc.load_gather(
            tailtab, [jnp.broadcast_to(row, (LANES,)), q * LANES + lane_iota])

    extract_chunk(ALIGNED_END, NENT, lambda e: e - ALIGNED_END, gatherT,
                  packs[0], poss[0], semW[0])

    drain_scatter(semW[0], packs[0])
    for k in range(1, 3):
        drain_scatter(semW[k], packs[k])
    for k in range(3):
        drain_stream(semS[k], bufs[k])


@functools.partial(
    pl.kernel,
    mesh=_mesh,
    out_type=jax.ShapeDtypeStruct((BATCH,), jnp.float32),
    scratch_types=[
        pltpu.VMEM((BPW,), jnp.int32),
        pltpu.VMEM((CH2, 2 * DIM), jnp.float32),
        pltpu.VMEM((CH2, 2 * DIM), jnp.float32),
        pltpu.VMEM((NREL, DIM), jnp.float32),
        pltpu.VMEM((BPW,), jnp.float32),
        pltpu.SemaphoreType.DMA,
    ],
    compiler_params=pltpu.CompilerParams(needs_layout_passes=False),
)
def _phase2(stage_hbm, rel_hbm, ridx_hbm, out_hbm,
            ridx_v, hrow, trow, rtab, outv, sem):
    wid = lax.axis_index("s") * 2 + lax.axis_index("c")
    base = wid * BPW

    pltpu.sync_copy(ridx_hbm.at[pl.ds(base, BPW)], ridx_v)
    pltpu.sync_copy(rel_hbm, rtab)

    lane_iota = lax.iota(jnp.int32, LANES)

    for c in range(NCH2):
        cbase = c * CH2
        cph = pltpu.async_copy(
            stage_hbm.at[pl.ds(base + cbase, CH2), :], hrow, sem)
        cpt = pltpu.async_copy(
            stage_hbm.at[pl.ds(BATCH + base + cbase, CH2), :], trow, sem)
        cph.wait()
        cpt.wait()

        def group_body(g, carry):
            rows = g * LANES + lane_iota
            rrows = ridx_v[pl.ds(cbase + g * LANES, LANES)]

            def dim_body(d, acc):
                cre = jnp.broadcast_to(d, (LANES,))
                cim = cre + HALF
                re_h = plsc.load_gather(hrow, [rows, cre])
                im_h = plsc.load_gather(hrow, [rows, cim])
                re_r = plsc.load_gather(rtab, [rrows, cre])
                im_r = plsc.load_gather(rtab, [rrows, cim])
                re_t = plsc.load_gather(trow, [rows, cre])
                im_t = plsc.load_gather(trow, [rows, cim])
                return acc + (re_h * (re_r * re_t + im_r * im_t)
                              + im_h * (re_r * im_t - im_r * re_t))

            acc = lax.fori_loop(0, HALF, dim_body,
                                jnp.zeros((LANES,), jnp.float32), unroll=4)
            outv[pl.ds(cbase + g * LANES, LANES)] = acc
            return carry

        lax.fori_loop(0, GROUPS2, group_body, 0)

    pltpu.sync_copy(outv, out_hbm.at[pl.ds(base, BPW)])


def kernel(head_index, relation_index, tail_index, time_index,
           entity_embedding, relation_embedding, time_embedding,
           aux_embedding):
    del time_index, time_embedding, aux_embedding
    hidx = head_index.astype(jnp.int32)
    tidx = tail_index.astype(jnp.int32)
    ridx = relation_index.astype(jnp.int32)
    tail_tbl = entity_embedding[ALIGNED_END:, :]
    staging = _phase1(entity_embedding.T, tail_tbl, hidx, tidx)
    return _phase2(staging, relation_embedding, ridx)


# speedup vs baseline: 1.2948x; 1.2948x over previous
import functools

import jax
import jax.numpy as jnp
from jax import lax
from jax.experimental import pallas as pl
from jax.experimental.pallas import tpu as pltpu
from jax.experimental.pallas import tpu_sc as plsc

BATCH = 16384
NENT = 1000000
DIM = 64
HALF = DIM // 2
LANES = 16
NREL = 230
NW = 32
RANGE = 31232
ALIGNED_END = 999936
NTAIL = NENT - ALIGNED_END
E = 768
NCHUNK = 42
MAXSTART = ALIGNED_END - E
LISTCAP = 1600
PACK = 64
NSTAGE = 2 * BATCH
STAGE = NSTAGE + PACK

BPW = BATCH // NW
NCH2 = 2
CH2 = BPW // NCH2
GROUPS2 = CH2 // LANES

_mesh = plsc.VectorSubcoreMesh(core_axis_name="c", subcore_axis_name="s")


@functools.partial(
    pl.kernel,
    mesh=_mesh,
    out_type=jax.ShapeDtypeStruct((STAGE, 2 * DIM), jnp.float32),
    scratch_types=[
        pltpu.VMEM((2048,), jnp.int32),
        pltpu.VMEM((LISTCAP,), jnp.int32),
        pltpu.VMEM((LISTCAP,), jnp.int32),
        [pltpu.VMEM((DIM, E), jnp.float32)] * 2,
        [pltpu.VMEM((PACK, 2 * DIM), jnp.float32)] * 2,
        [pltpu.VMEM((PACK,), jnp.int32)] * 2,
        pltpu.VMEM((PACK,), jnp.int32),
        pltpu.VMEM((NTAIL, DIM), jnp.float32),
        [pltpu.SemaphoreType.DMA] * 2,
        [pltpu.SemaphoreType.DMA] * 2,
    ],
    compiler_params=pltpu.CompilerParams(needs_layout_passes=False),
)
def _phase1(entT_hbm, tail_hbm, hidx_hbm, tidx_hbm, stage_hbm,
            piece, entlist, poslist, bufs, packs, poss, hitent, tailtab,
            semS, semW):
    wid = lax.axis_index("s") * 2 + lax.axis_index("c")
    lo = wid * RANGE
    hi = jnp.where(wid == NW - 1, NENT, lo + RANGE)
    lane_iota = lax.iota(jnp.int32, LANES)

    def chunk_start(c):
        return jnp.minimum(lo + c * E, MAXSTART)

    for k in range(2):
        pltpu.async_copy(
            entT_hbm.at[:, pl.ds(chunk_start(k), E)], bufs[k], semS[k])
    pltpu.sync_copy(tail_hbm, tailtab)

    def init_list(i, carry):
        entlist[pl.ds(i * LANES, LANES)] = jnp.full((LANES,), -1, jnp.int32)
        return carry
    lax.fori_loop(0, LISTCAP // LANES, init_list, 0)

    def scan_src(arr_hbm, pos0, cnt_in):
        def piece_body(p, cnt):
            pltpu.sync_copy(arr_hbm.at[pl.ds(p * 2048, 2048)], piece)

            def vec_body(i, cnt):
                v = piece[pl.ds(i * LANES, LANES)]
                m = (v >= lo) & (v < hi)
                cc = jnp.minimum(cnt, LISTCAP - LANES)
                plsc.store_compressed(entlist.at[pl.ds(cc, LANES)], v, mask=m)
                pos = pos0 + p * 2048 + i * LANES + lane_iota
                plsc.store_compressed(poslist.at[pl.ds(cc, LANES)], pos, mask=m)
                npop = plsc.all_reduce_population_count(m)
                return cnt + npop[0]

            return lax.fori_loop(0, 2048 // LANES, vec_body, cnt)
        return lax.fori_loop(0, BATCH // 2048, piece_body, cnt_in)

    cnt = scan_src(hidx_hbm, 0, jnp.int32(0))
    cnt = scan_src(tidx_hbm, BATCH, cnt)
    nvec = jnp.minimum((cnt + LANES - 1) // LANES, LISTCAP // LANES)

    def drain_stream(sem, buf):
        pltpu.make_async_copy(
            entT_hbm.at[:, pl.ds(0, E)], buf, sem).wait()

    def drain_scatter(sem, pack):
        pltpu.make_async_copy(
            stage_hbm.at[pl.ds(0, PACK), :], pack, sem).wait()

    def extract_chunk(e0, e1, col_of, src_gather, pack, posb, semWX):
        for q in range(PACK // LANES):
            posb[pl.ds(q * LANES, LANES)] = (
                NSTAGE + q * LANES + lane_iota)

        def list_body(i, hcnt):
            ev = entlist[pl.ds(i * LANES, LANES)]
            m = (ev >= e0) & (ev < e1)
            pv = poslist[pl.ds(i * LANES, LANES)]
            hc = jnp.minimum(hcnt, PACK - LANES)
            plsc.store_compressed(hitent.at[pl.ds(hc, LANES)], ev, mask=m)
            plsc.store_compressed(posb.at[pl.ds(hc, LANES)], pv, mask=m)
            return hcnt + plsc.all_reduce_population_count(m)[0]

        hcnt = lax.fori_loop(0, nvec, list_body, jnp.int32(0))
        hcnt = jnp.minimum(hcnt, PACK)

        def member_body(j, carry):
            jsplat = jnp.broadcast_to(j, (LANES,))
            e = plsc.load_gather(hitent, [jsplat])[0]
            col = col_of(e)
            for q in range(DIM // LANES):
                seg = src_gather(q, col)
                plsc.store_scatter(
                    pack, [jsplat, q * LANES + lane_iota], seg)
            return carry

        lax.fori_loop(0, hcnt, member_body, 0)
        pltpu.async_copy(pack, stage_hbm.at[posb], semWX)

    def pair_body(j, carry):
        for k in range(2):
            c = 2 * j + k
            e0 = lo + c * E
            s0 = chunk_start(c)
            drain_stream(semS[k], bufs[k])

            @pl.when(j > 0)
            def _():
                drain_scatter(semW[k], packs[k])

            def gk(q, col, _buf=bufs[k]):
                return plsc.load_gather(
                    _buf,
                    [q * LANES + lane_iota, jnp.broadcast_to(col, (LANES,))])

            extract_chunk(e0, e0 + E, lambda e: e - s0, gk,
                          packs[k], poss[k], semW[k])
            pltpu.async_copy(
                entT_hbm.at[:, pl.ds(chunk_start(c + 2), E)],
                bufs[k], semS[k])
        return carry

    lax.fori_loop(0, NCHUNK // 2, pair_body, 0)

    drain_scatter(semW[0], packs[0])

    def gatherT(q, row):
        return plsc.load_gather(
            tailtab, [jnp.broadcast_to(row, (LANES,)), q * LANES + lane_iota])

    extract_chunk(ALIGNED_END, NENT, lambda e: e - ALIGNED_END, gatherT,
                  packs[0], poss[0], semW[0])

    drain_scatter(semW[0], packs[0])
    drain_scatter(semW[1], packs[1])
    for k in range(2):
        drain_stream(semS[k], bufs[k])


@functools.partial(
    pl.kernel,
    mesh=_mesh,
    out_type=jax.ShapeDtypeStruct((BATCH,), jnp.float32),
    scratch_types=[
        pltpu.VMEM((BPW,), jnp.int32),
        pltpu.VMEM((CH2, 2 * DIM), jnp.float32),
        pltpu.VMEM((CH2, 2 * DIM), jnp.float32),
        pltpu.VMEM((NREL, DIM), jnp.float32),
        pltpu.VMEM((BPW,), jnp.float32),
        pltpu.SemaphoreType.DMA,
    ],
    compiler_params=pltpu.CompilerParams(needs_layout_passes=False),
)
def _phase2(stage_hbm, rel_hbm, ridx_hbm, out_hbm,
            ridx_v, hrow, trow, rtab, outv, sem):
    wid = lax.axis_index("s") * 2 + lax.axis_index("c")
    base = wid * BPW

    pltpu.sync_copy(ridx_hbm.at[pl.ds(base, BPW)], ridx_v)
    pltpu.sync_copy(rel_hbm, rtab)

    lane_iota = lax.iota(jnp.int32, LANES)

    for c in range(NCH2):
        cbase = c * CH2
        cph = pltpu.async_copy(
            stage_hbm.at[pl.ds(base + cbase, CH2), :], hrow, sem)
        cpt = pltpu.async_copy(
            stage_hbm.at[pl.ds(BATCH + base + cbase, CH2), :], trow, sem)
        cph.wait()
        cpt.wait()

        def group_body(g, carry):
            rows = g * LANES + lane_iota
            rrows = ridx_v[pl.ds(cbase + g * LANES, LANES)]

            def dim_body(d, acc):
                cre = jnp.broadcast_to(d, (LANES,))
                cim = cre + HALF
                re_h = plsc.load_gather(hrow, [rows, cre])
                im_h = plsc.load_gather(hrow, [rows, cim])
                re_r = plsc.load_gather(rtab, [rrows, cre])
                im_r = plsc.load_gather(rtab, [rrows, cim])
                re_t = plsc.load_gather(trow, [rows, cre])
                im_t = plsc.load_gather(trow, [rows, cim])
                return acc + (re_h * (re_r * re_t + im_r * im_t)
                              + im_h * (re_r * im_t - im_r * re_t))

            acc = lax.fori_loop(0, HALF, dim_body,
                                jnp.zeros((LANES,), jnp.float32), unroll=4)
            outv[pl.ds(cbase + g * LANES, LANES)] = acc
            return carry

        lax.fori_loop(0, GROUPS2, group_body, 0)

    pltpu.sync_copy(outv, out_hbm.at[pl.ds(base, BPW)])


def kernel(head_index, relation_index, tail_index, time_index,
           entity_embedding, relation_embedding, time_embedding,
           aux_embedding):
    del time_index, time_embedding, aux_embedding
    hidx = head_index.astype(jnp.int32)
    tidx = tail_index.astype(jnp.int32)
    ridx = relation_index.astype(jnp.int32)
    tail_tbl = entity_embedding[ALIGNED_END:, :]
    staging = _phase1(entity_embedding.T, tail_tbl, hidx, tidx)
    return _phase2(staging, relation_embedding, ridx)
